# baseline (speedup 1.0000x reference)
import functools

import jax
import jax.numpy as jnp
from jax import lax
from jax.experimental import pallas as pl
from jax.experimental.pallas import tpu as pltpu
from jax.experimental.pallas import tpu_sc as plsc

N_NODES = 10000
E_EDGES = 160000
D_FEAT = 256
D_EDGE = 16
D_OUT = D_FEAT + D_EDGE


def _node_matmul_body(x_ref, w_ref, b_ref, o_ref):
    acc = lax.dot_general(x_ref[...], w_ref[...], (((1,), (1,)), ((), ())),
                          preferred_element_type=jnp.float32)
    o_ref[...] = acc + 0.5 * b_ref[...]


def _node_matmul(x, Wx, bx):
    blk = 1000
    return pl.pallas_call(
        _node_matmul_body,
        grid=(N_NODES // blk,),
        in_specs=[
            pl.BlockSpec((blk, D_FEAT), lambda i: (i, 0)),
            pl.BlockSpec((D_FEAT, D_FEAT), lambda i: (0, 0)),
            pl.BlockSpec((1, D_FEAT), lambda i: (0, 0)),
        ],
        out_specs=pl.BlockSpec((blk, D_FEAT), lambda i: (i, 0)),
        out_shape=jax.ShapeDtypeStruct((N_NODES, D_FEAT), jnp.float32),
    )(x, Wx, bx.reshape(1, D_FEAT))


def _edge_matmul_body(a_ref, w_ref, b_ref, o_ref):
    acc = lax.dot_general(a_ref[...], w_ref[...], (((1,), (1,)), ((), ())),
                          preferred_element_type=jnp.float32)
    o_ref[...] = acc + b_ref[...]


def _edge_matmul(edge_attr, We, be):
    blk = 8000
    return pl.pallas_call(
        _edge_matmul_body,
        grid=(E_EDGES // blk,),
        in_specs=[
            pl.BlockSpec((blk, D_EDGE), lambda i: (i, 0)),
            pl.BlockSpec((D_EDGE, D_EDGE), lambda i: (0, 0)),
            pl.BlockSpec((1, D_EDGE), lambda i: (0, 0)),
        ],
        out_specs=pl.BlockSpec((blk, D_EDGE), lambda i: (i, 0)),
        out_shape=jax.ShapeDtypeStruct((E_EDGES, D_EDGE), jnp.float32),
    )(edge_attr, We, be.reshape(1, D_EDGE))


_NC, _NS, _LANES = 2, 16, 16
_NW = _NC * _NS
_EPW = E_EDGES // _NW
_CHUNK = 40
_NCHUNK = _EPW // _CHUNK


def _sc_gather_concat(y, idx_i, idx_j, e_lin):
    mesh = plsc.VectorSubcoreMesh(core_axis_name="c", subcore_axis_name="s")

    @functools.partial(
        pl.kernel,
        mesh=mesh,
        out_type=jax.ShapeDtypeStruct((E_EDGES, D_OUT), jnp.float32),
        scratch_types=[
            pltpu.VMEM((_EPW,), jnp.int32),
            pltpu.VMEM((_EPW,), jnp.int32),
            pltpu.VMEM((_CHUNK, D_FEAT), jnp.float32),
            pltpu.VMEM((_CHUNK, D_FEAT), jnp.float32),
            pltpu.VMEM((_CHUNK, D_FEAT), jnp.float32),
            pltpu.VMEM((_CHUNK, D_FEAT), jnp.float32),
            pltpu.VMEM((_CHUNK, D_EDGE), jnp.float32),
            pltpu.VMEM((_CHUNK, D_EDGE), jnp.float32),
            pltpu.VMEM((_CHUNK, D_OUT), jnp.float32),
            pltpu.SemaphoreType.DMA,
            pltpu.SemaphoreType.DMA,
            pltpu.SemaphoreType.DMA,
            pltpu.SemaphoreType.DMA,
            pltpu.SemaphoreType.DMA,
            pltpu.SemaphoreType.DMA,
        ],
    )
    def body(y_hbm, ii_hbm, jj_hbm, e_hbm, out_hbm,
             ii_v, jj_v, a0, a1, b0, b1, e0, e1, o_v,
             sa0, sa1, sb0, sb1, se0, se1):
        a_v, b_v, e_v = (a0, a1), (b0, b1), (e0, e1)
        sa, sb, se = (sa0, sa1), (sb0, sb1), (se0, se1)
        wid = lax.axis_index("s") * _NC + lax.axis_index("c")
        base = wid * _EPW
        pltpu.sync_copy(ii_hbm.at[pl.ds(base, _EPW)], ii_v)
        pltpu.sync_copy(jj_hbm.at[pl.ds(base, _EPW)], jj_v)

        def start(c, p):
            off = c * _CHUNK
            pltpu.async_copy(y_hbm.at[ii_v.at[pl.ds(off, _CHUNK)]], a_v[p], sa[p])
            pltpu.async_copy(y_hbm.at[jj_v.at[pl.ds(off, _CHUNK)]], b_v[p], sb[p])
            pltpu.async_copy(e_hbm.at[pl.ds(base + off, _CHUNK)], e_v[p], se[p])

        def wait_set(p):
            pltpu.make_async_copy(y_hbm.at[pl.ds(0, _CHUNK)], a_v[p], sa[p]).wait()
            pltpu.make_async_copy(y_hbm.at[pl.ds(0, _CHUNK)], b_v[p], sb[p]).wait()
            pltpu.make_async_copy(e_hbm.at[pl.ds(0, _CHUNK)], e_v[p], se[p]).wait()

        def process(c, p):
            def row_body(r, rcarry):
                for k in range(D_FEAT // _LANES):
                    sl = pl.ds(k * _LANES, _LANES)
                    o_v[r, sl] = a_v[p][r, sl] + b_v[p][r, sl]
                o_v[r, pl.ds(D_FEAT, _LANES)] = e_v[p][r, :]
                return rcarry

            lax.fori_loop(0, _CHUNK, row_body, 0)
            pltpu.sync_copy(o_v, out_hbm.at[pl.ds(base + c * _CHUNK, _CHUNK)])

        start(0, 0)

        def outer(i, carry):
            g = 2 * i
            for b in (0, 1):
                c = g + b
                start(c + 1, 1 - b)
                wait_set(b)
                process(c, b)
            return carry

        lax.fori_loop(0, (_NCHUNK - 1) // 2, outer, 0)
        wait_set(0)
        process(_NCHUNK - 1, 0)

    return body(y, idx_i, idx_j, e_lin)


def kernel(x, edge_index, edge_attr, Wx, bx, We, be):
    ei = edge_index.astype(jnp.int32)
    y = _node_matmul(x, Wx, bx)
    e_lin = _edge_matmul(edge_attr, We, be)
    return _sc_gather_concat(y, ei[0], ei[1], e_lin)
